# baseline (speedup 1.0000x reference)
import jax
import jax.numpy as jnp
from jax import lax
from jax.experimental import pallas as pl
from jax.experimental.pallas import tpu as pltpu
from jax.experimental.pallas import tpu_sc as plsc

NC = 2
NS = 16
NW = NC * NS

BATCH = 16384
HIST = 50
EMBED_DIM = 32
B_PER_W = BATCH // NW
HP = 56
CB = 8
N_CHUNKS = B_PER_W // CB


def _gather_body(idx_hbm, table_hbm, out_hbm, idx_v, rows_a, rows_b, gs_a,
                 gs_b, ws_a, ws_b):
    wid = lax.axis_index("s") * NC + lax.axis_index("c")
    base = wid * B_PER_W
    pltpu.sync_copy(idx_hbm.at[pl.ds(base * HP, B_PER_W * HP)], idx_v)
    rows = (rows_a, rows_b)
    gsem = (gs_a, gs_b)
    wsem = (ws_a, ws_b)

    def fire_gathers(c, buf):
        for j in range(CB):
            pltpu.async_copy(
                table_hbm.at[idx_v.at[pl.ds((c * CB + j) * HP, HIST)]],
                rows[buf].at[j], gsem[buf])

    def drain_gathers(c, buf):
        pltpu.make_async_copy(
            out_hbm.at[pl.ds(base + c * CB, CB)], rows[buf], gsem[buf]).wait()

    def write_chunk(c, buf):
        pltpu.async_copy(
            rows[buf], out_hbm.at[pl.ds(base + c * CB, CB)], wsem[buf])

    def wait_write(c, buf):
        pltpu.make_async_copy(
            rows[buf], out_hbm.at[pl.ds(base + c * CB, CB)], wsem[buf]).wait()

    fire_gathers(0, 0)

    @pl.loop(0, N_CHUNKS, step=2)
    def _outer(c0):
        for b in (0, 1):
            c = c0 + b
            nxt = 1 - b

            @pl.when(c + 1 < N_CHUNKS)
            def _fire():
                @pl.when(c >= 1)
                def _w():
                    wait_write(c - 1, nxt)
                fire_gathers(c + 1, nxt)

            drain_gathers(c, b)
            write_chunk(c, b)

    wait_write(N_CHUNKS - 2, 0)
    wait_write(N_CHUNKS - 1, 1)


def kernel(inputs, table):
    idx = jnp.pad(inputs.astype(jnp.int32), ((0, 0), (0, HP - HIST))).reshape(-1)
    mesh = plsc.VectorSubcoreMesh(
        core_axis_name="c", subcore_axis_name="s", num_cores=NC, num_subcores=NS
    )
    out = pl.kernel(
        _gather_body,
        out_type=jax.ShapeDtypeStruct((BATCH, HIST, EMBED_DIM), jnp.float32),
        mesh=mesh,
        compiler_params=pltpu.CompilerParams(use_tc_tiling_on_sc=False),
        scratch_types=[
            pltpu.VMEM((B_PER_W * HP,), jnp.int32),
            pltpu.VMEM((CB, HIST, EMBED_DIM), jnp.float32),
            pltpu.VMEM((CB, HIST, EMBED_DIM), jnp.float32),
            pltpu.SemaphoreType.DMA,
            pltpu.SemaphoreType.DMA,
            pltpu.SemaphoreType.DMA,
            pltpu.SemaphoreType.DMA,
        ],
    )(idx, table)
    return out
